# baseline (speedup 1.0000x reference)
import functools
import jax
import jax.numpy as jnp
from jax import lax
from jax.experimental import pallas as pl
from jax.experimental.pallas import tpu as pltpu
from jax.experimental.pallas import tpu_sc as plsc

CHANNELS = 768
TOPK = 384
TEMP = 1.0
NB = 16
H = 56
W = 56
ROWS = NB * H * W
R_BLK = 3584
N_RBLK = ROWS // R_BLK
HALF = CHANNELS // 2
CH_PER_SUB = HALF // 16

_MESH = plsc.VectorSubcoreMesh(core_axis_name="c", subcore_axis_name="s")


@functools.partial(
    pl.kernel,
    out_type=(
        jax.ShapeDtypeStruct((CHANNELS,), jnp.float32),
        jax.ShapeDtypeStruct((32,), jnp.int32),
    ),
    mesh=_MESH,
    compiler_params=pltpu.CompilerParams(needs_layout_passes=False),
    scratch_types=[
        pltpu.VMEM((CHANNELS,), jnp.float32),
        pltpu.VMEM((32,), jnp.float32),
        pltpu.VMEM((16,), jnp.float32),
        pltpu.VMEM_SHARED((16, 16), jnp.float32),
        pltpu.VMEM((16, 16), jnp.float32),
        pltpu.VMEM((16,), jnp.int32),
    ],
)
def _sc_mask(logits_hbm, m_hbm, meta_hbm,
             lg_v, m_v, cnt_v, shared, tmp_v, meta_v):
    cid = lax.axis_index("c")
    sid = lax.axis_index("s")
    base = cid * HALF + sid * CH_PER_SUB

    pltpu.sync_copy(logits_hbm, lg_v)

    lane = lax.iota(jnp.int32, 16)
    i1 = base + lane
    i2 = base + 16 + lane
    v1 = plsc.load_gather(lg_v, [i1])
    v2 = plsc.load_gather(lg_v, [i2])

    def body(j, acc):
        a1, a2 = acc
        jv = jnp.full((16,), 0, jnp.int32) + j
        vj = plsc.load_gather(lg_v, [jv])
        b1 = (vj > v1) | ((vj == v1) & (jv < i1))
        b2 = (vj > v2) | ((vj == v2) & (jv < i2))
        return (a1 + b1.astype(jnp.int32), a2 + b2.astype(jnp.int32))

    zero = jnp.zeros((16,), jnp.int32)
    rank1, rank2 = lax.fori_loop(0, CHANNELS, body, (zero, zero))

    hard1 = (rank1 < TOPK).astype(jnp.float32)
    hard2 = (rank2 < TOPK).astype(jnp.float32)
    s1 = 1.0 / (1.0 + jnp.exp(-v1 / TEMP))
    s2 = 1.0 / (1.0 + jnp.exp(-v2 / TEMP))
    m_v[pl.ds(0, 16)] = (hard1 - s1) + s1
    m_v[pl.ds(16, 16)] = (hard2 - s2) + s2
    pltpu.sync_copy(m_v.at[pl.ds(0, CH_PER_SUB)],
                    m_hbm.at[pl.ds(base, CH_PER_SUB)])

    kept = jnp.sum(hard1) + jnp.sum(jnp.where(lane < 8, hard2, 0.0))
    cnt_v[...] = jnp.zeros((16,), jnp.float32) + kept
    pltpu.sync_copy(cnt_v, shared.at[sid])
    plsc.subcore_barrier()

    @pl.when(sid == 0)
    def _():
        pltpu.sync_copy(shared, tmp_v)
        total = jnp.zeros((16,), jnp.float32)
        for k in range(16):
            total = total + tmp_v[k]
        flag = (total > 0.0).astype(jnp.int32)
        meta_v[...] = jnp.where(lane == 0, flag, 0)
        pltpu.sync_copy(meta_v, meta_hbm.at[pl.ds(16 * cid, 16)])


def _gate_kernel(meta_ref, z0_ref, z1_ref, m_ref, out_ref):
    del meta_ref
    out_ref[:, :HALF] = z0_ref[...] * m_ref[0, :HALF][None, :]
    out_ref[:, HALF:] = z1_ref[...] * m_ref[0, HALF:][None, :]


def kernel(z, logits):
    zt = z.transpose(0, 2, 3, 1).reshape(ROWS, CHANNELS)
    m_out, meta = _sc_mask(logits)
    m2 = m_out.reshape(1, CHANNELS)
    meta2 = meta.reshape(1, 32)

    def z0_map(r, meta):
        return (jnp.where(meta[0, 0] > 0, r, N_RBLK - 1), 0)

    def z1_map(r, meta):
        return (jnp.where(meta[0, 16] > 0, r, N_RBLK - 1), 1)

    grid_spec = pltpu.PrefetchScalarGridSpec(
        num_scalar_prefetch=1,
        grid=(N_RBLK,),
        in_specs=[
            pl.BlockSpec((R_BLK, HALF), z0_map),
            pl.BlockSpec((R_BLK, HALF), z1_map),
            pl.BlockSpec((1, CHANNELS), lambda r, meta: (0, 0)),
        ],
        out_specs=pl.BlockSpec((R_BLK, CHANNELS), lambda r, meta: (r, 0)),
    )
    out = pl.pallas_call(
        _gate_kernel,
        grid_spec=grid_spec,
        out_shape=jax.ShapeDtypeStruct((ROWS, CHANNELS), jnp.float32),
    )(meta2, zt, zt, m2)
    return out.reshape(NB, H, W, CHANNELS).transpose(0, 3, 1, 2)


# speedup vs baseline: 1.2869x; 1.2869x over previous
import jax
import jax.numpy as jnp
from jax.experimental import pallas as pl
from jax.experimental.pallas import tpu as pltpu

CHANNELS = 768
TOPK = 384
TEMP = 1.0
C_BLK = 128
N_CBLK = CHANNELS // C_BLK
NB = 16
H = 56
W = 56
ROWS = NB * H * W
R_BLK = 3584
N_RBLK = ROWS // R_BLK


def _mask_kernel(logits_ref, m_ref, meta_ref):
    lg = logits_ref[0, :]
    col = lg[None, :]
    row = lg[:, None]
    i_idx = jax.lax.broadcasted_iota(jnp.int32, (CHANNELS, CHANNELS), 0)
    j_idx = jax.lax.broadcasted_iota(jnp.int32, (CHANNELS, CHANNELS), 1)
    beats = (col > row) | ((col == row) & (j_idx < i_idx))
    rank = jnp.sum(beats.astype(jnp.int32), axis=1)
    hard = (rank < TOPK).astype(jnp.float32)
    soft = jax.nn.sigmoid(lg / TEMP)
    m = (hard - soft) + soft
    m_ref[0, :] = m

    wact = (jnp.sum(hard.reshape(2, CHANNELS // 2), axis=1) > 0).astype(jnp.int32)
    lane = jax.lax.broadcasted_iota(jnp.int32, (1, 128), 1)[0]
    meta = (jnp.where(lane == 0, wact[0], 0)
            + jnp.where(lane == 1, wact[1], 0))
    meta_ref[0, :] = meta


HALF = CHANNELS // 2


def _gate_kernel(meta_ref, z0_ref, z1_ref, m_ref, out_ref):
    del meta_ref
    out_ref[:, :HALF] = z0_ref[...] * m_ref[0, :HALF][None, :]
    out_ref[:, HALF:] = z1_ref[...] * m_ref[0, HALF:][None, :]


def kernel(z, logits):
    zt = z.transpose(0, 2, 3, 1).reshape(ROWS, CHANNELS)
    m_out, meta = pl.pallas_call(
        _mask_kernel,
        out_shape=(
            jax.ShapeDtypeStruct((1, CHANNELS), jnp.float32),
            jax.ShapeDtypeStruct((1, 128), jnp.int32),
        ),
    )(logits.reshape(1, CHANNELS))

    def z0_map(r, meta):
        return (jnp.where(meta[0, 0] > 0, r, N_RBLK - 1), 0)

    def z1_map(r, meta):
        return (jnp.where(meta[0, 1] > 0, r, N_RBLK - 1), 1)

    grid_spec = pltpu.PrefetchScalarGridSpec(
        num_scalar_prefetch=1,
        grid=(N_RBLK,),
        in_specs=[
            pl.BlockSpec((R_BLK, HALF), z0_map),
            pl.BlockSpec((R_BLK, HALF), z1_map),
            pl.BlockSpec((1, CHANNELS), lambda r, meta: (0, 0)),
        ],
        out_specs=pl.BlockSpec((R_BLK, CHANNELS), lambda r, meta: (r, 0)),
    )
    out = pl.pallas_call(
        _gate_kernel,
        grid_spec=grid_spec,
        out_shape=jax.ShapeDtypeStruct((ROWS, CHANNELS), jnp.float32),
    )(meta, zt, zt, m_out)
    return out.reshape(NB, H, W, CHANNELS).transpose(0, 3, 1, 2)
